# baseline (speedup 1.0000x reference)
import jax
import jax.numpy as jnp
from jax.experimental import pallas as pl
from jax.experimental.pallas import tpu as pltpu


def kernel(x, adj_sparse, eps, W1, b1, W2, b2):
    N, D_IN = x.shape
    D_HID = W1.shape[0]
    D_OUT = W2.shape[0]

    BM = 200
    nm = N // (2 * BM)

    w1t = W1.T
    w2t = W2.T
    b1r = b1.reshape(1, D_HID)
    b2r = b2.reshape(1, D_OUT)
    epsr = eps.reshape(1, 1)

    def body(a0_ref, a1_ref, x_ref, eps_ref, w1_ref, b1_ref, w2_ref, b2_ref,
             o_ref):
        i = pl.program_id(0)
        scale = 1.0 + eps_ref[0, 0]
        for half, a_ref in ((0, a0_ref), (1, a1_ref)):
            h = jnp.dot(a_ref[...], x_ref[...],
                        preferred_element_type=jnp.float32)
            xm = x_ref[pl.ds((2 * i + half) * BM, BM), :]
            h = h + scale * xm
            h1 = jnp.maximum(
                jnp.dot(h, w1_ref[...],
                        preferred_element_type=jnp.float32) + b1_ref[...],
                0.0)
            o_ref[pl.ds(half * BM, BM), :] = jnp.dot(
                h1, w2_ref[...],
                preferred_element_type=jnp.float32) + b2_ref[...]

    return pl.pallas_call(
        body,
        grid=(nm,),
        in_specs=[
            pl.BlockSpec((BM, N), lambda i: (2 * i, 0)),
            pl.BlockSpec((BM, N), lambda i: (2 * i + 1, 0)),
            pl.BlockSpec((N, D_IN), lambda i: (0, 0)),
            pl.BlockSpec((1, 1), lambda i: (0, 0)),
            pl.BlockSpec((D_IN, D_HID), lambda i: (0, 0)),
            pl.BlockSpec((1, D_HID), lambda i: (0, 0)),
            pl.BlockSpec((D_HID, D_OUT), lambda i: (0, 0)),
            pl.BlockSpec((1, D_OUT), lambda i: (0, 0)),
        ],
        out_specs=pl.BlockSpec((2 * BM, D_OUT), lambda i: (i, 0)),
        out_shape=jax.ShapeDtypeStruct((N, D_OUT), jnp.float32),
        compiler_params=pltpu.CompilerParams(
            dimension_semantics=("arbitrary",)),
    )(adj_sparse, adj_sparse, x, epsr, w1t, b1r, w2t, b2r)
